# initial kernel scaffold (unmeasured)
import jax
import jax.numpy as jnp
from jax import lax
from jax.experimental import pallas as pl
from jax.experimental.pallas import tpu as pltpu


def kernel(x, dest):
    m_rows, n_cols = x.shape
    my_x = lax.axis_index("x")

    keep_mask = dest == my_x
    k = jnp.sum(keep_mask.astype(jnp.int32))
    n = m_rows - k
    perm = jnp.argsort(keep_mask, stable=True)
    x_arr = jnp.take(x, perm, axis=0)

    def body(x_ref, out_ref, send_sem, recv_sem):
        mx = lax.axis_index("x")
        my = lax.axis_index("y")
        other = 1 - mx

        barrier_sem = pltpu.get_barrier_semaphore()
        pl.semaphore_signal(
            barrier_sem,
            inc=1,
            device_id=(other, my),
            device_id_type=pl.DeviceIdType.MESH,
        )
        pl.semaphore_wait(barrier_sem, 1)

        rdma = pltpu.make_async_remote_copy(
            src_ref=x_ref,
            dst_ref=out_ref,
            send_sem=send_sem,
            recv_sem=recv_sem,
            device_id=(other, my),
            device_id_type=pl.DeviceIdType.MESH,
        )
        rdma.start()
        rdma.wait()

    recv = pl.pallas_call(
        body,
        out_shape=jax.ShapeDtypeStruct((m_rows, n_cols), x.dtype),
        in_specs=[pl.BlockSpec(memory_space=pltpu.VMEM)],
        out_specs=pl.BlockSpec(memory_space=pltpu.VMEM),
        scratch_shapes=[
            pltpu.SemaphoreType.DMA,
            pltpu.SemaphoreType.DMA,
        ],
        compiler_params=pltpu.CompilerParams(collective_id=0),
    )(x_arr)

    j = jnp.arange(m_rows)
    comb = jnp.concatenate([x_arr, recv], axis=0)
    idx0 = jnp.where(j < k, n + j, m_rows + (j - k))
    idx1 = jnp.where(j < n, m_rows + j, j)
    idx = jnp.where(my_x == 0, idx0, idx1)
    return jnp.take(comb, idx, axis=0)


# baseline (device time: 67073 ns/iter reference)
import jax
import jax.numpy as jnp
from jax import lax
from jax.experimental import pallas as pl
from jax.experimental.pallas import tpu as pltpu

M_ROWS = 2048
CHUNK = 128
N_SEG = M_ROWS // CHUNK
TAIL_SEM = 15
N_REPAIR = 16


def kernel(x, dest):
    m_rows, n_cols = x.shape

    def body(x_ref, dest_ref, dest2d_ref, out_ref, send_buf,
             send_sems, recv_sems, rep_i, rep_p):
        mx = lax.axis_index("x")
        my = lax.axis_index("y")
        other = 1 - mx

        barrier_sem = pltpu.get_barrier_semaphore()
        pl.semaphore_signal(
            barrier_sem,
            inc=1,
            device_id=(other, my),
            device_id_type=pl.DeviceIdType.MESH,
        )
        pl.semaphore_wait(barrier_sem, 1)

        k = jnp.sum((dest2d_ref[...] == mx).astype(jnp.int32))
        n = m_rows - k
        own_base = jnp.where(mx == 0, 0, n)
        rbv = jnp.where(mx == 0, 0, k)
        delta = rbv % 8
        a0 = pl.multiple_of(rbv - delta, 8)
        lv = pl.multiple_of((rbv + n + 7) // 8 * 8 - a0, 8)
        cv = (lv + CHUNK - 1) // CHUNK
        b_in = jnp.where(mx == 0, k, 0)
        a0_in = b_in - b_in % 8
        l_in = (b_in + n + 7) // 8 * 8 - a0_in
        cv_in = (l_in + CHUNK - 1) // CHUNK

        def chunk_rdma(src_lo, dst_lo, sem_idx):
            return pltpu.make_async_remote_copy(
                src_ref=send_buf.at[pl.ds(src_lo, CHUNK)],
                dst_ref=out_ref.at[pl.ds(dst_lo, CHUNK)],
                send_sem=send_sems.at[sem_idx],
                recv_sem=recv_sems.at[sem_idx],
                device_id=(other, my),
                device_id_type=pl.DeviceIdType.MESH,
            )

        def stage_row(i, carry):
            kc, sc, rc = carry
            is_keep = dest_ref[i] == mx
            p_keep = own_base + kc
            p_send = delta + sc

            @pl.when(is_keep)
            def _():
                out_ref[pl.ds(p_keep, 1), :] = x_ref[pl.ds(i, 1), :]

            @pl.when(jnp.logical_not(is_keep))
            def _():
                send_buf[pl.ds(p_send, 1), :] = x_ref[pl.ds(i, 1), :]

            in_margin = is_keep & (
                ((p_keep >= a0_in) & (p_keep < b_in))
                | ((p_keep >= b_in + n) & (p_keep < a0_in + l_in))
            )

            @pl.when(in_margin)
            def _():
                rep_i[rc] = i
                rep_p[rc] = p_keep

            w = is_keep.astype(jnp.int32)
            return (kc + w, sc + 1 - w, rc + in_margin.astype(jnp.int32))

        carry = (jnp.int32(0), jnp.int32(0), jnp.int32(0))
        issued = jnp.int32(0)
        for s in range(N_SEG):
            carry = lax.fori_loop(
                s * CHUNK, (s + 1) * CHUNK, stage_row, carry, unroll=4
            )
            sc = carry[1]
            ready = jnp.minimum((delta + sc) // CHUNK, cv - 1)
            for c in range(N_SEG - 1):
                @pl.when((issued == c) & (ready > c))
                def _(c=c):
                    chunk_rdma(
                        c * CHUNK, pl.multiple_of(a0 + c * CHUNK, 8), c
                    ).start()
            issued = jnp.maximum(issued, ready)

        for c in range(N_SEG - 1):
            @pl.when((c >= issued) & (c < cv - 1))
            def _(c=c):
                chunk_rdma(
                    c * CHUNK, pl.multiple_of(a0 + c * CHUNK, 8), c
                ).start()

        chunk_rdma(
            pl.multiple_of(lv - CHUNK, 8),
            pl.multiple_of(a0 + lv - CHUNK, 8),
            TAIL_SEM,
        ).start()

        for c in range(N_SEG - 1):
            @pl.when(c < cv - 1)
            def _(c=c):
                chunk_rdma(c * CHUNK, 0, c).wait_send()

            @pl.when(c < cv_in - 1)
            def _(c=c):
                chunk_rdma(c * CHUNK, 0, c).wait_recv()

        chunk_rdma(0, 0, TAIL_SEM).wait_send()
        chunk_rdma(0, 0, TAIL_SEM).wait_recv()

        rc_final = carry[2]
        for r in range(N_REPAIR):
            @pl.when(r < rc_final)
            def _(r=r):
                out_ref[pl.ds(rep_p[r], 1), :] = x_ref[pl.ds(rep_i[r], 1), :]

    return pl.pallas_call(
        body,
        out_shape=jax.ShapeDtypeStruct((m_rows, n_cols), x.dtype),
        in_specs=[
            pl.BlockSpec(memory_space=pltpu.VMEM),
            pl.BlockSpec(memory_space=pltpu.SMEM),
            pl.BlockSpec(memory_space=pltpu.VMEM),
        ],
        out_specs=pl.BlockSpec(memory_space=pltpu.VMEM),
        scratch_shapes=[
            pltpu.VMEM((m_rows, n_cols), x.dtype),
            pltpu.SemaphoreType.DMA((N_SEG,)),
            pltpu.SemaphoreType.DMA((N_SEG,)),
            pltpu.SMEM((N_REPAIR,), jnp.int32),
            pltpu.SMEM((N_REPAIR,), jnp.int32),
        ],
        compiler_params=pltpu.CompilerParams(collective_id=0),
    )(x, dest, dest.reshape(N_SEG, CHUNK))


# device time: 66171 ns/iter; 1.0136x vs baseline; 1.0136x over previous
import jax
import jax.numpy as jnp
from jax import lax
from jax.experimental import pallas as pl
from jax.experimental.pallas import tpu as pltpu

M_ROWS = 2048
CHUNK = 128
N_SEG = M_ROWS // CHUNK
TAIL_SEM = 15
N_REPAIR = 16


def kernel(x, dest):
    m_rows, n_cols = x.shape

    def body(x_ref, dest_ref, dest2d_ref, out_ref, send_buf,
             send_sems, recv_sems, rep_i, rep_p):
        mx = lax.axis_index("x")
        my = lax.axis_index("y")
        other = 1 - mx

        barrier_sem = pltpu.get_barrier_semaphore()
        pl.semaphore_signal(
            barrier_sem,
            inc=1,
            device_id=(other, my),
            device_id_type=pl.DeviceIdType.MESH,
        )
        pl.semaphore_wait(barrier_sem, 1)

        k = jnp.sum((dest2d_ref[...] == mx).astype(jnp.int32))
        n = m_rows - k
        own_base = jnp.where(mx == 0, 0, n)
        rbv = jnp.where(mx == 0, 0, k)
        delta = rbv % 8
        a0 = pl.multiple_of(rbv - delta, 8)
        lv = pl.multiple_of((rbv + n + 7) // 8 * 8 - a0, 8)
        cv = (lv + CHUNK - 1) // CHUNK
        b_in = jnp.where(mx == 0, k, 0)
        a0_in = b_in - b_in % 8
        l_in = (b_in + n + 7) // 8 * 8 - a0_in
        cv_in = (l_in + CHUNK - 1) // CHUNK

        def chunk_rdma(src_lo, dst_lo, sem_idx):
            return pltpu.make_async_remote_copy(
                src_ref=send_buf.at[pl.ds(src_lo, CHUNK)],
                dst_ref=out_ref.at[pl.ds(dst_lo, CHUNK)],
                send_sem=send_sems.at[sem_idx],
                recv_sem=recv_sems.at[sem_idx],
                device_id=(other, my),
                device_id_type=pl.DeviceIdType.MESH,
            )

        def stage_send(i, sc):
            is_send = dest_ref[i] != mx

            @pl.when(is_send)
            def _():
                send_buf[pl.ds(delta + sc, 1), :] = x_ref[pl.ds(i, 1), :]

            return sc + is_send.astype(jnp.int32)

        sc = jnp.int32(0)
        issued = jnp.int32(0)
        for s in range(N_SEG):
            sc = lax.fori_loop(
                s * CHUNK, (s + 1) * CHUNK, stage_send, sc, unroll=8
            )
            ready = jnp.minimum((delta + sc) // CHUNK, cv - 1)
            for c in range(N_SEG - 1):
                @pl.when((issued == c) & (ready > c))
                def _(c=c):
                    chunk_rdma(
                        c * CHUNK, pl.multiple_of(a0 + c * CHUNK, 8), c
                    ).start()
            issued = jnp.maximum(issued, ready)

        for c in range(N_SEG - 1):
            @pl.when((c >= issued) & (c < cv - 1))
            def _(c=c):
                chunk_rdma(
                    c * CHUNK, pl.multiple_of(a0 + c * CHUNK, 8), c
                ).start()

        chunk_rdma(
            pl.multiple_of(lv - CHUNK, 8),
            pl.multiple_of(a0 + lv - CHUNK, 8),
            TAIL_SEM,
        ).start()

        def stage_keep(i, carry):
            kc, rc = carry
            is_keep = dest_ref[i] == mx
            p_keep = own_base + kc

            @pl.when(is_keep)
            def _():
                out_ref[pl.ds(p_keep, 1), :] = x_ref[pl.ds(i, 1), :]

            in_margin = is_keep & (
                ((p_keep >= a0_in) & (p_keep < b_in))
                | ((p_keep >= b_in + n) & (p_keep < a0_in + l_in))
            )

            @pl.when(in_margin)
            def _():
                rep_i[rc] = i
                rep_p[rc] = p_keep

            return (kc + is_keep.astype(jnp.int32),
                    rc + in_margin.astype(jnp.int32))

        kc, rc_final = lax.fori_loop(
            0, m_rows, stage_keep, (jnp.int32(0), jnp.int32(0)), unroll=8
        )

        for c in range(N_SEG - 1):
            @pl.when(c < cv - 1)
            def _(c=c):
                chunk_rdma(c * CHUNK, 0, c).wait_send()

            @pl.when(c < cv_in - 1)
            def _(c=c):
                chunk_rdma(c * CHUNK, 0, c).wait_recv()

        chunk_rdma(0, 0, TAIL_SEM).wait_send()
        chunk_rdma(0, 0, TAIL_SEM).wait_recv()

        for r in range(N_REPAIR):
            @pl.when(r < rc_final)
            def _(r=r):
                out_ref[pl.ds(rep_p[r], 1), :] = x_ref[pl.ds(rep_i[r], 1), :]

    return pl.pallas_call(
        body,
        out_shape=jax.ShapeDtypeStruct((m_rows, n_cols), x.dtype),
        in_specs=[
            pl.BlockSpec(memory_space=pltpu.VMEM),
            pl.BlockSpec(memory_space=pltpu.SMEM),
            pl.BlockSpec(memory_space=pltpu.VMEM),
        ],
        out_specs=pl.BlockSpec(memory_space=pltpu.VMEM),
        scratch_shapes=[
            pltpu.VMEM((m_rows, n_cols), x.dtype),
            pltpu.SemaphoreType.DMA((N_SEG,)),
            pltpu.SemaphoreType.DMA((N_SEG,)),
            pltpu.SMEM((N_REPAIR,), jnp.int32),
            pltpu.SMEM((N_REPAIR,), jnp.int32),
        ],
        compiler_params=pltpu.CompilerParams(collective_id=0),
    )(x, dest, dest.reshape(N_SEG, CHUNK))


# device time: 60854 ns/iter; 1.1022x vs baseline; 1.0874x over previous
import jax
import jax.numpy as jnp
from jax import lax
from jax.experimental import pallas as pl
from jax.experimental.pallas import tpu as pltpu

M_ROWS = 2048
SEG = 64
CHUNK = 64
N_SEG = M_ROWS // SEG
N_SEMS = 32
TAIL_SEM = 31
N_REPAIR = 16


def kernel(x, dest):
    m_rows, n_cols = x.shape

    def body(x_ref, dest_ref, dest2d_ref, out_ref, send_buf,
             send_sems, recv_sems, rep_i, rep_p):
        mx = lax.axis_index("x")
        my = lax.axis_index("y")
        other = 1 - mx

        barrier_sem = pltpu.get_barrier_semaphore()
        pl.semaphore_signal(
            barrier_sem,
            inc=1,
            device_id=(other, my),
            device_id_type=pl.DeviceIdType.MESH,
        )

        k = jnp.sum((dest2d_ref[...] == mx).astype(jnp.int32))
        n = m_rows - k
        own_base = jnp.where(mx == 0, 0, n)
        rbv = jnp.where(mx == 0, 0, k)
        delta = rbv % 8
        a0 = pl.multiple_of(rbv - delta, 8)
        lv = pl.multiple_of((rbv + n + 7) // 8 * 8 - a0, 8)
        cv = (lv + CHUNK - 1) // CHUNK
        b_in = jnp.where(mx == 0, k, 0)
        a0_in = b_in - b_in % 8
        l_in = (b_in + n + 7) // 8 * 8 - a0_in
        cv_in = (l_in + CHUNK - 1) // CHUNK

        pl.semaphore_wait(barrier_sem, 1)

        def chunk_rdma(src_lo, dst_lo, sem_idx):
            return pltpu.make_async_remote_copy(
                src_ref=send_buf.at[pl.ds(src_lo, CHUNK)],
                dst_ref=out_ref.at[pl.ds(dst_lo, CHUNK)],
                send_sem=send_sems.at[sem_idx],
                recv_sem=recv_sems.at[sem_idx],
                device_id=(other, my),
                device_id_type=pl.DeviceIdType.MESH,
            )

        def stage_send(i, sc):
            is_send = dest_ref[i] != mx

            @pl.when(is_send)
            def _():
                send_buf[pl.ds(delta + sc, 1), :] = x_ref[pl.ds(i, 1), :]

            return sc + is_send.astype(jnp.int32)

        sc = jnp.int32(0)
        issued = jnp.int32(0)
        for s in range(N_SEG):
            sc = lax.fori_loop(
                s * SEG, (s + 1) * SEG, stage_send, sc, unroll=8
            )
            ready = jnp.minimum((delta + sc) // CHUNK, cv - 1)

            @pl.when(ready > issued)
            def _(issued=issued):
                chunk_rdma(
                    pl.multiple_of(issued * CHUNK, 8),
                    pl.multiple_of(a0 + issued * CHUNK, 8),
                    issued,
                ).start()

            issued = jnp.maximum(issued, ready)

        def issue_left(c, _):
            @pl.when((c >= issued) & (c < cv - 1))
            def _():
                chunk_rdma(
                    pl.multiple_of(c * CHUNK, 8),
                    pl.multiple_of(a0 + c * CHUNK, 8),
                    c,
                ).start()
            return 0

        lax.fori_loop(0, N_SEMS - 1, issue_left, 0)

        chunk_rdma(
            pl.multiple_of(lv - CHUNK, 8),
            pl.multiple_of(a0 + lv - CHUNK, 8),
            TAIL_SEM,
        ).start()

        def stage_keep(i, carry):
            kc, rc = carry
            is_keep = dest_ref[i] == mx
            p_keep = own_base + kc

            @pl.when(is_keep)
            def _():
                out_ref[pl.ds(p_keep, 1), :] = x_ref[pl.ds(i, 1), :]

            in_margin = is_keep & (
                ((p_keep >= a0_in) & (p_keep < b_in))
                | ((p_keep >= b_in + n) & (p_keep < a0_in + l_in))
            )

            @pl.when(in_margin)
            def _():
                rep_i[rc] = i
                rep_p[rc] = p_keep

            return (kc + is_keep.astype(jnp.int32),
                    rc + in_margin.astype(jnp.int32))

        kc, rc_final = lax.fori_loop(
            0, m_rows, stage_keep, (jnp.int32(0), jnp.int32(0)), unroll=8
        )

        def drain_recv(c, _):
            @pl.when(c < cv_in - 1)
            def _():
                chunk_rdma(0, 0, c).wait_recv()
            return 0

        lax.fori_loop(0, N_SEMS - 1, drain_recv, 0)
        chunk_rdma(0, 0, TAIL_SEM).wait_recv()

        for r in range(N_REPAIR):
            @pl.when(r < rc_final)
            def _(r=r):
                out_ref[pl.ds(rep_p[r], 1), :] = x_ref[pl.ds(rep_i[r], 1), :]

        def drain_send(c, _):
            @pl.when(c < cv - 1)
            def _():
                chunk_rdma(0, 0, c).wait_send()
            return 0

        lax.fori_loop(0, N_SEMS - 1, drain_send, 0)
        chunk_rdma(0, 0, TAIL_SEM).wait_send()

    return pl.pallas_call(
        body,
        out_shape=jax.ShapeDtypeStruct((m_rows, n_cols), x.dtype),
        in_specs=[
            pl.BlockSpec(memory_space=pltpu.VMEM),
            pl.BlockSpec(memory_space=pltpu.SMEM),
            pl.BlockSpec(memory_space=pltpu.VMEM),
        ],
        out_specs=pl.BlockSpec(memory_space=pltpu.VMEM),
        scratch_shapes=[
            pltpu.VMEM((M_ROWS, 1024), jnp.float32),
            pltpu.SemaphoreType.DMA((N_SEMS,)),
            pltpu.SemaphoreType.DMA((N_SEMS,)),
            pltpu.SMEM((N_REPAIR,), jnp.int32),
            pltpu.SMEM((N_REPAIR,), jnp.int32),
        ],
        compiler_params=pltpu.CompilerParams(collective_id=0),
    )(x, dest, dest.reshape(N_SEG, SEG))
